# baseline (speedup 1.0000x reference)
import jax
import jax.numpy as jnp
from jax.experimental import pallas as pl
from jax.experimental.pallas import tpu as pltpu

DIM = 4096
NUM_EXPERTS = 64
EPSILON = 1e-06
BLOCK_N = 256


def _gate_block(x_ref, w_ref, b_ref, out_ref, masked_ref, denom_ref):
    i = pl.program_id(0)
    nblk = pl.num_programs(0)
    logits = jax.lax.dot_general(
        x_ref[:], w_ref[:], (((1,), (1,)), ((), ())),
        preferred_element_type=jnp.float32) + b_ref[:]
    m = jnp.max(logits, axis=1, keepdims=True)
    e = jnp.exp(logits - m)
    s = jnp.sum(e, axis=1, keepdims=True)
    idx = jnp.argmax(logits, axis=1)
    cols = jax.lax.broadcasted_iota(jnp.int32, logits.shape, 1)
    masked = jnp.where(cols == idx[:, None], 1.0 / s, 0.0)
    masked_ref[pl.ds(i * BLOCK_N, BLOCK_N), :] = masked
    part = jnp.sum(masked, axis=0, keepdims=True)

    @pl.when(i == 0)
    def _init():
        denom_ref[:] = part

    @pl.when(i > 0)
    def _acc():
        denom_ref[:] += part

    @pl.when(i == nblk - 1)
    def _finish():
        n = masked_ref.shape[0]
        scale = float(n) / (denom_ref[:] + EPSILON)
        out_ref[:] = masked_ref[:] * scale


def kernel(x, W, b):
    batch_size, seq_len, dim = x.shape
    n = batch_size * seq_len
    xf = x.reshape(n, dim)
    b2 = b.reshape(1, NUM_EXPERTS)
    nblk = n // BLOCK_N

    out = pl.pallas_call(
        _gate_block,
        grid=(nblk,),
        in_specs=[
            pl.BlockSpec((BLOCK_N, dim), lambda i: (i, 0)),
            pl.BlockSpec((NUM_EXPERTS, dim), lambda i: (0, 0)),
            pl.BlockSpec((1, NUM_EXPERTS), lambda i: (0, 0)),
        ],
        out_specs=pl.BlockSpec((n, NUM_EXPERTS), lambda i: (0, 0)),
        out_shape=jax.ShapeDtypeStruct((n, NUM_EXPERTS), jnp.float32),
        scratch_shapes=[
            pltpu.VMEM((n, NUM_EXPERTS), jnp.float32),
            pltpu.VMEM((1, NUM_EXPERTS), jnp.float32),
        ],
    )(xf, W, b2)

    return out.reshape(batch_size, seq_len, NUM_EXPERTS)


# speedup vs baseline: 1.1762x; 1.1762x over previous
import jax
import jax.numpy as jnp
from jax.experimental import pallas as pl
from jax.experimental.pallas import tpu as pltpu

DIM = 4096
NUM_EXPERTS = 64
EPSILON = 1e-06
BLOCK_N = 512
HALF = 4096


def _gate_one(x, w, b):
    logits = jax.lax.dot_general(
        x, w, (((1,), (1,)), ((), ())),
        preferred_element_type=jnp.float32) + b
    m = jnp.max(logits, axis=1, keepdims=True)
    e = jnp.exp(logits - m)
    s = jnp.sum(e, axis=1, keepdims=True)
    idx = jnp.argmax(logits, axis=1)
    cols = jax.lax.broadcasted_iota(jnp.int32, logits.shape, 1)
    return jnp.where(cols == idx[:, None], 1.0 / s, 0.0)


def _gate_block(xa_ref, xb_ref, w_ref, b_ref, out_ref, masked_ref, denom_ref):
    i = pl.program_id(0)
    nblk = pl.num_programs(0)
    w = w_ref[:]
    b = b_ref[:]
    ma = _gate_one(xa_ref[0], w, b)
    mb = _gate_one(xb_ref[0], w, b)
    masked_ref[pl.ds(i * BLOCK_N, BLOCK_N), :] = ma
    masked_ref[pl.ds(HALF + i * BLOCK_N, BLOCK_N), :] = mb
    part = jnp.sum(ma, axis=0, keepdims=True) + jnp.sum(mb, axis=0, keepdims=True)

    @pl.when(i == 0)
    def _init():
        denom_ref[:] = part

    @pl.when(i > 0)
    def _acc():
        denom_ref[:] += part

    @pl.when(i == nblk - 1)
    def _finish():
        n = masked_ref.shape[0]
        scale = float(n) / (denom_ref[:] + EPSILON)
        out_ref[:] = masked_ref[:] * scale


def kernel(x, W, b):
    batch_size, seq_len, dim = x.shape
    n = batch_size * seq_len
    xr = x.reshape(2, n // 2, dim)
    b2 = b.reshape(1, NUM_EXPERTS)
    nblk = (n // 2) // BLOCK_N

    out = pl.pallas_call(
        _gate_block,
        grid=(nblk,),
        in_specs=[
            pl.BlockSpec((1, BLOCK_N, dim), lambda i: (0, i, 0)),
            pl.BlockSpec((1, BLOCK_N, dim), lambda i: (1, i, 0)),
            pl.BlockSpec((NUM_EXPERTS, dim), lambda i: (0, 0)),
            pl.BlockSpec((1, NUM_EXPERTS), lambda i: (0, 0)),
        ],
        out_specs=pl.BlockSpec((n, NUM_EXPERTS), lambda i: (0, 0)),
        out_shape=jax.ShapeDtypeStruct((n, NUM_EXPERTS), jnp.float32),
        scratch_shapes=[
            pltpu.VMEM((n, NUM_EXPERTS), jnp.float32),
            pltpu.VMEM((1, NUM_EXPERTS), jnp.float32),
        ],
    )(xr, xr, W, b2)

    return out.reshape(batch_size, seq_len, NUM_EXPERTS)
